# baseline (speedup 1.0000x reference)
import functools

import jax
import jax.numpy as jnp
from jax import lax
from jax.experimental import pallas as pl
from jax.experimental.pallas import tpu as pltpu
from jax.experimental.pallas import tpu_sc as plsc

D_MODEL = 64
SCALE = 8.0

NC, NS, L = 2, 16, 16
NW = NC * NS
BB = 128
FH = D_MODEL // 8
OPITCH = BB + 1
TCHUNK = 4096
NBUF = 4


def _transpose_tc(table_t, nblk):
    d, v = table_t.shape
    h = TCHUNK // 2

    def body(inb, outb):
        t = inb[...].T.astype(jnp.bfloat16)
        outb[:, 0:D_MODEL] = t[0:h]
        outb[:, D_MODEL:128] = t[h:TCHUNK]

    return pl.pallas_call(
        body,
        grid=(nblk,),
        in_specs=[pl.BlockSpec((d, TCHUNK), lambda i: (0, i))],
        out_specs=pl.BlockSpec((h, 128), lambda i: (i, 0)),
        out_shape=jax.ShapeDtypeStruct((nblk * h, 128), jnp.bfloat16),
        compiler_params=pltpu.CompilerParams(
            dimension_semantics=("parallel",)
        ),
    )(table_t)


def _make_gather_sc(T, NB, VPAD):
    mesh = plsc.VectorSubcoreMesh(core_axis_name="c", subcore_axis_name="s")

    @functools.partial(
        pl.kernel,
        out_type=jax.ShapeDtypeStruct((T, FH, NB, 8, BB), jnp.float32),
        mesh=mesh,
        compiler_params=pltpu.CompilerParams(
            use_tc_tiling_on_sc=False, needs_layout_passes=False
        ),
        scratch_types=(
            [pltpu.VMEM((T, BB), jnp.int32)]
            + [pltpu.VMEM((BB, D_MODEL), jnp.bfloat16)] * NBUF
            + [pltpu.VMEM((D_MODEL, OPITCH), jnp.float32)] * NBUF
            + [pltpu.SemaphoreType.DMA] * (2 * NBUF)
        ),
    )
    def emb(xi_hbm, tab_hbm, out_hbm, idx_v, *bufs):
        gbufs = bufs[0:NBUF]
        obufs = bufs[NBUF : 2 * NBUF]
        gsems = bufs[2 * NBUF : 3 * NBUF]
        osems = bufs[3 * NBUF : 4 * NBUF]
        w = lax.axis_index("s") * NC + lax.axis_index("c")
        pltpu.sync_copy(xi_hbm.at[:, w], idx_v)

        jrows = [
            (lax.iota(jnp.int32, L) * 2 + par + j2 * 32, j2, par)
            for j2 in range(2)
            for par in range(2)
        ]

        def issue(t, buf, sem):
            pltpu.async_copy(tab_hbm.at[idx_v.at[t]], buf, sem)

        def drain_g(buf, sem):
            pltpu.make_async_copy(tab_hbm.at[pl.ds(0, BB)], buf, sem).wait()

        def drain_o(buf, sem):
            @pl.loop(0, FH)
            def _(fh):
                pltpu.make_async_copy(
                    buf.at[pl.ds(0, 8), pl.ds(0, BB)], out_hbm.at[0, 0, 0], sem
                ).wait()

        for p in range(NBUF - 1):
            issue(p, gbufs[p], gsems[p])

        @pl.loop(0, T, step=NBUF)
        def _units(t0):
            for b in range(NBUF):
                t = t0 + b
                gb, ob = gbufs[b], obufs[b]

                nb = (b + NBUF - 1) % NBUF

                @pl.when(t + NBUF - 1 < T)
                def _():
                    issue(t + NBUF - 1, gbufs[nb], gsems[nb])

                drain_g(gb, gsems[b])

                @pl.when(t >= NBUF)
                def _():
                    drain_o(ob, osems[b])

                @plsc.parallel_loop(0, BB, unroll=4)
                def _tok(bc):
                    row = gb.at[bc]
                    cols = jnp.zeros((L,), jnp.int32) + bc
                    for j2 in range(2):
                        packed = row[pl.ds(j2 * 32, 32)]
                        a, bb_ = plsc.unpack(
                            packed, format=plsc.PackFormat.INTERLEAVED
                        )
                        for rows_vec, jj2, par in jrows:
                            if jj2 != j2:
                                continue
                            vals = (a if par == 0 else bb_) * SCALE
                            plsc.store_scatter(ob, [rows_vec, cols], vals)

                @pl.loop(0, FH, unroll=2)
                def _ship(fh):
                    pltpu.async_copy(
                        ob.at[pl.ds(fh * 8, 8), pl.ds(0, BB)],
                        out_hbm.at[t, fh, w],
                        osems[b],
                    )

        for p in range(NBUF):
            drain_o(obufs[p], osems[p])

    return emb


def kernel(x, table):
    BT, T = x.shape
    V, D = table.shape
    NB = BT // BB
    nblk = (V + TCHUNK - 1) // TCHUNK
    vpad = nblk * TCHUNK

    tab_lin = _transpose_tc(table.T, nblk).reshape(vpad, D)

    half_shift = TCHUNK.bit_length() - 2
    xw = x.astype(jnp.int32)
    u = jnp.bitwise_and(xw, TCHUNK - 1)
    s = jnp.bitwise_and(jnp.right_shift(xw, half_shift), 1)
    xj = xw + u - (TCHUNK - 1) * s
    xi = xj.T.reshape(T, NB, BB)

    out6 = _make_gather_sc(T, NB, vpad)(xi, tab_lin)
    return out6.transpose(2, 4, 0, 1, 3).reshape(BT, T, D)


# speedup vs baseline: 1.9459x; 1.9459x over previous
import functools

import jax
import jax.numpy as jnp
from jax import lax
from jax.experimental import pallas as pl
from jax.experimental.pallas import tpu as pltpu
from jax.experimental.pallas import tpu_sc as plsc

D_MODEL = 64
SCALE = 8.0

NC, NS, L = 2, 16, 16
NW = NC * NS
BB = 128
FH = D_MODEL // 8
OPITCH = BB + 1
TCHUNK = 8192
NBUF = 4


def _transpose_tc(table_t, nblk):
    d, v = table_t.shape
    h = TCHUNK // 2

    def body(inb, outb):
        t = inb[...].T
        outb[:, 0:D_MODEL] = t[0:h]
        outb[:, D_MODEL:128] = t[h:TCHUNK]

    return pl.pallas_call(
        body,
        grid=(nblk,),
        in_specs=[pl.BlockSpec((d, TCHUNK), lambda i: (0, i))],
        out_specs=pl.BlockSpec((h, 128), lambda i: (i, 0)),
        out_shape=jax.ShapeDtypeStruct((nblk * h, 128), jnp.float32),
        compiler_params=pltpu.CompilerParams(
            dimension_semantics=("parallel",)
        ),
    )(table_t)


def _make_gather_sc(T, NB, VPAD):
    mesh = plsc.VectorSubcoreMesh(core_axis_name="c", subcore_axis_name="s")

    @functools.partial(
        pl.kernel,
        out_type=jax.ShapeDtypeStruct((T, FH, NB, 8, BB), jnp.float32),
        mesh=mesh,
        compiler_params=pltpu.CompilerParams(
            use_tc_tiling_on_sc=False, needs_layout_passes=False
        ),
        scratch_types=(
            [pltpu.VMEM((T, BB), jnp.int32)]
            + [pltpu.VMEM((BB, D_MODEL), jnp.float32)] * NBUF
            + [pltpu.VMEM((D_MODEL, OPITCH), jnp.float32)] * NBUF
            + [pltpu.SemaphoreType.DMA] * (2 * NBUF)
        ),
    )
    def emb(xi_hbm, tab_hbm, out_hbm, idx_v, *bufs):
        gbufs = bufs[0:NBUF]
        obufs = bufs[NBUF : 2 * NBUF]
        gsems = bufs[2 * NBUF : 3 * NBUF]
        osems = bufs[3 * NBUF : 4 * NBUF]
        w = lax.axis_index("s") * NC + lax.axis_index("c")
        pltpu.sync_copy(xi_hbm.at[:, w], idx_v)

        jrows = [lax.iota(jnp.int32, L) + j * L for j in range(D_MODEL // L)]

        def issue(t, buf, sem):
            pltpu.async_copy(tab_hbm.at[idx_v.at[t]], buf, sem)

        def drain_g(buf, sem):
            pltpu.make_async_copy(tab_hbm.at[pl.ds(0, BB)], buf, sem).wait()

        def drain_o(buf, sem):
            @pl.loop(0, FH)
            def _(fh):
                pltpu.make_async_copy(
                    buf.at[pl.ds(0, 8), pl.ds(0, BB)], out_hbm.at[0, 0, 0], sem
                ).wait()

        for p in range(NBUF - 1):
            issue(p, gbufs[p], gsems[p])

        @pl.loop(0, T, step=NBUF)
        def _units(t0):
            for b in range(NBUF):
                t = t0 + b
                gb, ob = gbufs[b], obufs[b]

                nb = (b + NBUF - 1) % NBUF

                @pl.when(t + NBUF - 1 < T)
                def _():
                    issue(t + NBUF - 1, gbufs[nb], gsems[nb])

                drain_g(gb, gsems[b])

                @pl.when(t >= NBUF)
                def _():
                    drain_o(ob, osems[b])

                @plsc.parallel_loop(0, BB, unroll=4)
                def _tok(bc):
                    row = gb.at[bc]
                    cols = jnp.zeros((L,), jnp.int32) + bc
                    for j in range(D_MODEL // L):
                        vals = row[pl.ds(j * L, L)] * SCALE
                        plsc.store_scatter(ob, [jrows[j], cols], vals)

                @pl.loop(0, FH, unroll=2)
                def _ship(fh):
                    pltpu.async_copy(
                        ob.at[pl.ds(fh * 8, 8), pl.ds(0, BB)],
                        out_hbm.at[t, fh, w],
                        osems[b],
                    )

        for p in range(NBUF):
            drain_o(obufs[p], osems[p])

    return emb


def kernel(x, table):
    BT, T = x.shape
    V, D = table.shape
    NB = BT // BB
    nblk = (V + TCHUNK - 1) // TCHUNK
    vpad = nblk * TCHUNK

    tab_lin = _transpose_tc(table.T, nblk).reshape(vpad, D)

    half_shift = TCHUNK.bit_length() - 2
    xw = x.astype(jnp.int32)
    u = jnp.bitwise_and(xw, TCHUNK - 1)
    s = jnp.bitwise_and(jnp.right_shift(xw, half_shift), 1)
    xj = xw + u - (TCHUNK - 1) * s
    xi = xj.T.reshape(T, NB, BB)

    out6 = _make_gather_sc(T, NB, vpad)(xi, tab_lin)
    return out6.transpose(2, 4, 0, 1, 3).reshape(BT, T, D)


# speedup vs baseline: 2.1118x; 1.0853x over previous
import functools

import jax
import jax.numpy as jnp
from jax import lax
from jax.experimental import pallas as pl
from jax.experimental.pallas import tpu as pltpu
from jax.experimental.pallas import tpu_sc as plsc

D_MODEL = 64
SCALE = 8.0

NC, NS, L = 2, 16, 16
NW = NC * NS
BB = 128
FH = D_MODEL // 8
OPITCH = BB + 1
TCHUNK = 16384
NBUF = 4


def _transpose_tc(table_t, nblk):
    d, v = table_t.shape
    h = TCHUNK // 2

    def body(inb, outb):
        t = inb[...].T
        outb[:, 0:D_MODEL] = t[0:h]
        outb[:, D_MODEL:128] = t[h:TCHUNK]

    return pl.pallas_call(
        body,
        grid=(nblk,),
        in_specs=[pl.BlockSpec((d, TCHUNK), lambda i: (0, i))],
        out_specs=pl.BlockSpec((h, 128), lambda i: (i, 0)),
        out_shape=jax.ShapeDtypeStruct((nblk * h, 128), jnp.float32),
        compiler_params=pltpu.CompilerParams(
            dimension_semantics=("parallel",)
        ),
    )(table_t)


def _make_gather_sc(T, NB, VPAD):
    mesh = plsc.VectorSubcoreMesh(core_axis_name="c", subcore_axis_name="s")

    @functools.partial(
        pl.kernel,
        out_type=jax.ShapeDtypeStruct((T, FH, NB, 8, BB), jnp.float32),
        mesh=mesh,
        compiler_params=pltpu.CompilerParams(
            use_tc_tiling_on_sc=False, needs_layout_passes=False
        ),
        scratch_types=(
            [pltpu.VMEM((T, BB), jnp.int32)]
            + [pltpu.VMEM((BB, D_MODEL), jnp.float32)] * NBUF
            + [pltpu.VMEM((D_MODEL, OPITCH), jnp.float32)] * NBUF
            + [pltpu.SemaphoreType.DMA] * (2 * NBUF)
        ),
    )
    def emb(xi_hbm, tab_hbm, out_hbm, idx_v, *bufs):
        gbufs = bufs[0:NBUF]
        obufs = bufs[NBUF : 2 * NBUF]
        gsems = bufs[2 * NBUF : 3 * NBUF]
        osems = bufs[3 * NBUF : 4 * NBUF]
        w = lax.axis_index("s") * NC + lax.axis_index("c")
        pltpu.sync_copy(xi_hbm.at[:, w], idx_v)

        jrows = [lax.iota(jnp.int32, L) + j * L for j in range(D_MODEL // L)]

        def issue(t, buf, sem):
            pltpu.async_copy(tab_hbm.at[idx_v.at[t]], buf, sem)

        def drain_g(buf, sem):
            pltpu.make_async_copy(tab_hbm.at[pl.ds(0, BB)], buf, sem).wait()

        def drain_o(buf, sem):
            @pl.loop(0, FH)
            def _(fh):
                pltpu.make_async_copy(
                    buf.at[pl.ds(0, 8), pl.ds(0, BB)], out_hbm.at[0, 0, 0], sem
                ).wait()

        for p in range(NBUF - 1):
            issue(p, gbufs[p], gsems[p])

        @pl.loop(0, T, step=NBUF)
        def _units(t0):
            for b in range(NBUF):
                t = t0 + b
                gb, ob = gbufs[b], obufs[b]

                nb = (b + NBUF - 1) % NBUF

                @pl.when(t + NBUF - 1 < T)
                def _():
                    issue(t + NBUF - 1, gbufs[nb], gsems[nb])

                drain_g(gb, gsems[b])

                @pl.when(t >= NBUF)
                def _():
                    drain_o(ob, osems[b])

                @plsc.parallel_loop(0, BB, unroll=4)
                def _tok(bc):
                    row = gb.at[bc]
                    cols = jnp.zeros((L,), jnp.int32) + bc
                    for j in range(D_MODEL // L):
                        vals = row[pl.ds(j * L, L)] * SCALE
                        plsc.store_scatter(ob, [jrows[j], cols], vals)

                @pl.loop(0, FH, unroll=2)
                def _ship(fh):
                    pltpu.async_copy(
                        ob.at[pl.ds(fh * 8, 8), pl.ds(0, BB)],
                        out_hbm.at[t, fh, w],
                        osems[b],
                    )

        for p in range(NBUF):
            drain_o(obufs[p], osems[p])

    return emb


def kernel(x, table):
    BT, T = x.shape
    V, D = table.shape
    NB = BT // BB
    nblk = (V + TCHUNK - 1) // TCHUNK
    vpad = nblk * TCHUNK

    tab_lin = _transpose_tc(table.T, nblk).reshape(vpad, D)

    half_shift = TCHUNK.bit_length() - 2
    xw = x.astype(jnp.int32)
    u = jnp.bitwise_and(xw, TCHUNK - 1)
    s = jnp.bitwise_and(jnp.right_shift(xw, half_shift), 1)
    xj = xw + u - (TCHUNK - 1) * s
    xi = xj.T.reshape(T, NB, BB)

    out6 = _make_gather_sc(T, NB, vpad)(xi, tab_lin)
    return out6.transpose(2, 4, 0, 1, 3).reshape(BT, T, D)
